# baseline (speedup 1.0000x reference)
import functools

import jax
import jax.numpy as jnp
from jax import lax
from jax.experimental import pallas as pl
from jax.experimental.pallas import tpu as pltpu
from jax.experimental.pallas import tpu_sc as plsc

NUM_STATES = 128
NUM_OBS = 100000
BATCH = 16384

LANES = 16
CHUNK = 8192
N_CHUNKS = BATCH // CHUNK
ROWS_PER_TILE = NUM_STATES // 32


def _lse_body(m_ref, o_ref):
    x = m_ref[...]
    mx = jnp.max(x, axis=1, keepdims=True)
    s = jnp.sum(jnp.exp(x - mx), axis=1, keepdims=True)
    o_ref[...] = mx + jnp.log(s)


def _lse(m):
    return pl.pallas_call(
        _lse_body,
        grid=(NUM_STATES // 8,),
        in_specs=[pl.BlockSpec((8, NUM_OBS), lambda i: (i, 0))],
        out_specs=pl.BlockSpec((8, 1), lambda i: (i, 0)),
        out_shape=jax.ShapeDtypeStruct((NUM_STATES, 1), jnp.float32),
    )(m)


def _make_gather():
    mesh = plsc.VectorSubcoreMesh(core_axis_name="c", subcore_axis_name="s")

    @functools.partial(
        pl.kernel,
        mesh=mesh,
        compiler_params=pltpu.CompilerParams(needs_layout_passes=False),
        out_type=jax.ShapeDtypeStruct((NUM_STATES, BATCH), jnp.float32),
        scratch_types=[
            pltpu.VMEM((NUM_OBS,), jnp.float32),
            pltpu.VMEM((BATCH,), jnp.int32),
            pltpu.VMEM((CHUNK,), jnp.float32),
        ],
    )
    def gather_k(m_hbm, obs_hbm, out_hbm, row_v, idx_v, g_v):
        wid = lax.axis_index("s") * 2 + lax.axis_index("c")

        pltpu.sync_copy(obs_hbm, idx_v)

        for r in range(ROWS_PER_TILE):
            state = wid * ROWS_PER_TILE + r
            pltpu.sync_copy(m_hbm.at[state], row_v)

            for c in range(N_CHUNKS):
                def body(k, carry):
                    off = pl.multiple_of(c * CHUNK + k * LANES, LANES)
                    iv = idx_v[pl.ds(off, LANES)]
                    g = plsc.load_gather(row_v, [iv])
                    goff = pl.multiple_of(k * LANES, LANES)
                    g_v[pl.ds(goff, LANES)] = g
                    return carry

                lax.fori_loop(0, CHUNK // LANES, body, 0)
                pltpu.sync_copy(g_v, out_hbm.at[state, pl.ds(c * CHUNK, CHUNK)])

    return gather_k


_gather = _make_gather()


_TB = 2048


def _tsub_body(g_ref, l_ref, o_ref):
    g = g_ref[...]
    l = l_ref[...]
    o_ref[...] = g.T - l.T


def _tsub(gathered, lse):
    return pl.pallas_call(
        _tsub_body,
        grid=(BATCH // _TB,),
        in_specs=[
            pl.BlockSpec((NUM_STATES, _TB), lambda i: (0, i)),
            pl.BlockSpec((NUM_STATES, 1), lambda i: (0, 0)),
        ],
        out_specs=pl.BlockSpec((_TB, NUM_STATES), lambda i: (i, 0)),
        out_shape=jax.ShapeDtypeStruct((BATCH, NUM_STATES), jnp.float32),
    )(gathered, lse)


def kernel(observation, emission_logits_matrix):
    obs = observation.astype(jnp.int32)
    lse = _lse(emission_logits_matrix)
    gat = _gather(emission_logits_matrix, obs)
    return _tsub(gat, lse)
